# baseline (speedup 1.0000x reference)
import functools

import jax
import jax.numpy as jnp
from jax import lax
from jax.experimental import pallas as pl
from jax.experimental.pallas import tpu as pltpu
from jax.experimental.pallas import tpu_sc as plsc

D = 1024
E = 8
K = 2
F_E = 1024
F_S = 2048
N = 2048

BLK = 128
P_R = N * K + E * BLK
NB = P_R // BLK

NEG = -1e30

SC_NC = 2
SC_NS = 16
SC_NW = SC_NC * SC_NS


def _gate_body(x_ref, wg_ref, s_ref, idx_ref, w_ref):
    x = x_ref[...]
    wg = wg_ref[...]
    s = jnp.dot(x, wg, preferred_element_type=jnp.float32)
    s_ref[...] = s
    lane = lax.broadcasted_iota(jnp.int32, s.shape, 1)
    sm = jnp.where(lane < E, s, NEG)
    m1 = jnp.max(sm, axis=1, keepdims=True)
    i1 = jnp.min(jnp.where(sm == m1, lane, 127), axis=1, keepdims=True)
    sm2 = jnp.where(lane == i1, NEG, sm)
    m2 = jnp.max(sm2, axis=1, keepdims=True)
    i2 = jnp.min(jnp.where(sm2 == m2, lane, 127), axis=1, keepdims=True)
    z = jnp.sum(jnp.exp(sm - m1), axis=1, keepdims=True)
    w1 = 1.0 / z
    w2 = jnp.exp(m2 - m1) / z
    idx_ref[...] = jnp.where(lane == 0, i1, jnp.where(lane == 1, i2, 0))
    w_ref[...] = jnp.where(lane == 0, w1, jnp.where(lane == 1, w2, 0.0))


def _gate(xf, wg_pad):
    tb = 256
    return pl.pallas_call(
        _gate_body,
        grid=(N // tb,),
        in_specs=[
            pl.BlockSpec((tb, D), lambda t: (t, 0)),
            pl.BlockSpec((D, 128), lambda t: (0, 0)),
        ],
        out_specs=[
            pl.BlockSpec((tb, 128), lambda t: (t, 0)),
            pl.BlockSpec((tb, 128), lambda t: (t, 0)),
            pl.BlockSpec((tb, 128), lambda t: (t, 0)),
        ],
        out_shape=[
            jax.ShapeDtypeStruct((N, 128), jnp.float32),
            jax.ShapeDtypeStruct((N, 128), jnp.int32),
            jax.ShapeDtypeStruct((N, 128), jnp.float32),
        ],
    )(xf, wg_pad)


def _route(e1, e2, w1, w2):
    ef = jnp.stack([e1, e2], axis=1).reshape(-1)
    wf = jnp.stack([w1, w2], axis=1).reshape(-1)
    onehot = (ef[:, None] == jnp.arange(E, dtype=jnp.int32)[None, :])
    counts = jnp.sum(onehot, axis=0, dtype=jnp.int32)
    pc = ((counts + BLK - 1) // BLK) * BLK
    ends = jnp.cumsum(pc)
    gs = ends - pc
    rank = jnp.cumsum(onehot.astype(jnp.int32), axis=0) - 1
    r = jnp.sum(rank * onehot, axis=1)
    pos = gs[ef] + r
    b_start = jnp.arange(NB, dtype=jnp.int32) * BLK
    be = jnp.sum(b_start[:, None] >= ends[None, :], axis=1).astype(jnp.int32)
    be = jnp.minimum(be, E - 1)
    return pos, wf, be


GATHER_PER_W = P_R // SC_NW
GATHER_CH = 16
G_CHUNKS = GATHER_PER_W // GATHER_CH
G_NBUF = 4


@functools.lru_cache(maxsize=None)
def _make_sc_gather():
    mesh = plsc.VectorSubcoreMesh(core_axis_name="c", subcore_axis_name="s")

    @functools.partial(
        pl.kernel,
        mesh=mesh,
        out_type=(
            jax.ShapeDtypeStruct((P_R, D // 2), jnp.int32),
            jax.ShapeDtypeStruct((P_R,), jnp.float32),
        ),
        scratch_types=[
            pltpu.VMEM((N * K,), jnp.int32),
            pltpu.VMEM((N * K,), jnp.float32),
            pltpu.VMEM((GATHER_PER_W,), jnp.int32),
            pltpu.VMEM((GATHER_PER_W,), jnp.float32),
            pltpu.VMEM((G_NBUF, GATHER_CH, D // 2), jnp.int32),
        ] + [pltpu.SemaphoreType.DMA] * (2 * G_NBUF),
        compiler_params=pltpu.CompilerParams(needs_layout_passes=False),
    )
    def sc_gather(x_hbm, pos_hbm, wf_hbm, out_hbm, sw_hbm,
                  pos_v, wf_v, gidx_v, sw_v, rows_v, *sems):
        gsem = sems[:G_NBUF]
        wsem = sems[G_NBUF:]
        wid = lax.axis_index("s") * SC_NC + lax.axis_index("c")
        base0 = wid * GATHER_PER_W
        pltpu.sync_copy(pos_hbm, pos_v)
        pltpu.sync_copy(wf_hbm, wf_v)
        lane = lax.iota(jnp.int32, 16)

        def init_body(i, _):
            sl = pl.ds(i * 16, 16)
            gidx_v[sl] = (base0 + i * 16 + lane) & (N - 1)
            sw_v[sl] = jnp.zeros((16,), jnp.float32)
            return 0

        lax.fori_loop(0, GATHER_PER_W // 16, init_body, 0)

        def scan_body(i, _):
            s = i * 16
            p = pos_v[pl.ds(s, 16)]
            w = wf_v[pl.ds(s, 16)]
            tok = (s + lane) >> 1
            li = p - base0
            m = (li >= 0) & (li < GATHER_PER_W)
            li = jnp.clip(li, 0, GATHER_PER_W - 1)
            plsc.store_scatter(gidx_v, [li], tok, mask=m)
            plsc.store_scatter(sw_v, [li], w, mask=m)
            return 0

        lax.fori_loop(0, (N * K) // 16, scan_body, 0)
        pltpu.sync_copy(sw_v, sw_hbm.at[pl.ds(base0, GATHER_PER_W)])

        def start_gather(c):
            buf = c % G_NBUF
            return pltpu.async_copy(
                x_hbm.at[gidx_v.at[pl.ds(c * GATHER_CH, GATHER_CH)]],
                rows_v.at[buf], gsem[buf])

        gh = {c: start_gather(c) for c in range(G_NBUF)}
        wh = {}
        for c in range(G_CHUNKS):
            buf = c % G_NBUF
            gh[c].wait()
            wh[c] = pltpu.async_copy(
                rows_v.at[buf],
                out_hbm.at[pl.ds(base0 + c * GATHER_CH, GATHER_CH)],
                wsem[buf])
            if c + G_NBUF < G_CHUNKS:
                wh[c].wait()
                gh[c + G_NBUF] = start_gather(c + G_NBUF)
        for c in range(max(0, G_CHUNKS - G_NBUF), G_CHUNKS):
            wh[c].wait()

    return sc_gather


def _sc_gather(xbf, pos, wf):
    return _make_sc_gather()(xbf, pos, wf)


def _ffn_body(be_ref, xg_ref, w1_ref, w2_ref, wp_ref, sw_ref, out_ref):
    xb = xg_ref[...]
    w1 = w1_ref[0].astype(jnp.bfloat16)
    w2 = w2_ref[0].astype(jnp.bfloat16)
    h1 = jnp.dot(xb, w1, preferred_element_type=jnp.float32)
    h2 = jnp.dot(xb, w2, preferred_element_type=jnp.float32)
    h = ((h1 * jax.nn.sigmoid(h1)) * h2).astype(jnp.bfloat16)
    wp = wp_ref[0].astype(jnp.bfloat16)
    y = jnp.dot(h, wp, preferred_element_type=jnp.float32)
    sw = sw_ref[...][:, 0:1]
    out_ref[...] = y * sw


def _ffn(be, xg, w1, w2, wp, sw_full):
    spec = pltpu.PrefetchScalarGridSpec(
        num_scalar_prefetch=1,
        grid=(NB,),
        in_specs=[
            pl.BlockSpec((BLK, D), lambda b, be: (b, 0)),
            pl.BlockSpec((1, D, F_E), lambda b, be: (be[b], 0, 0)),
            pl.BlockSpec((1, D, F_E), lambda b, be: (be[b], 0, 0)),
            pl.BlockSpec((1, F_E, D), lambda b, be: (be[b], 0, 0)),
            pl.BlockSpec((BLK, 128), lambda b, be: (b, 0)),
        ],
        out_specs=pl.BlockSpec((BLK, D), lambda b, be: (b, 0)),
    )
    return pl.pallas_call(
        _ffn_body,
        grid_spec=spec,
        out_shape=jax.ShapeDtypeStruct((P_R, D), jnp.float32),
    )(be, xg, w1, w2, wp, sw_full)


F_T = 512


def _shared_body(x_ref, ws1_ref, ws2_ref, wsp_ref, z_ref):
    f = pl.program_id(0)
    x = x_ref[...].astype(jnp.bfloat16)
    ws1 = ws1_ref[...].astype(jnp.bfloat16)
    ws2 = ws2_ref[...].astype(jnp.bfloat16)
    h1 = jnp.dot(x, ws1, preferred_element_type=jnp.float32)
    h2 = jnp.dot(x, ws2, preferred_element_type=jnp.float32)
    h = ((h1 * jax.nn.sigmoid(h1)) * h2).astype(jnp.bfloat16)
    wsp = wsp_ref[...].astype(jnp.bfloat16)
    part = jnp.dot(h, wsp, preferred_element_type=jnp.float32)

    @pl.when(f == 0)
    def _():
        z_ref[...] = part

    @pl.when(f != 0)
    def _():
        z_ref[...] = z_ref[...] + part


def _shared(xf, ws1, ws2, wsp):
    return pl.pallas_call(
        _shared_body,
        grid=(F_S // F_T,),
        in_specs=[
            pl.BlockSpec((N, D), lambda f: (0, 0)),
            pl.BlockSpec((D, F_T), lambda f: (0, f)),
            pl.BlockSpec((D, F_T), lambda f: (0, f)),
            pl.BlockSpec((F_T, D), lambda f: (f, 0)),
        ],
        out_specs=pl.BlockSpec((N, D), lambda f: (0, 0)),
        out_shape=jax.ShapeDtypeStruct((N, D), jnp.float32),
    )(xf, ws1, ws2, wsp)


COMB_PER_W = N // SC_NW
COMB_CH = 16
C_CHUNKS = COMB_PER_W // COMB_CH


@functools.lru_cache(maxsize=None)
def _make_sc_combine():
    mesh = plsc.VectorSubcoreMesh(core_axis_name="c", subcore_axis_name="s")

    @functools.partial(
        pl.kernel,
        mesh=mesh,
        out_type=jax.ShapeDtypeStruct((N, D), jnp.float32),
        scratch_types=[
            pltpu.VMEM((COMB_PER_W,), jnp.int32),
            pltpu.VMEM((COMB_PER_W,), jnp.int32),
            pltpu.VMEM((2, COMB_CH, D), jnp.float32),
            pltpu.VMEM((2, COMB_CH, D), jnp.float32),
            pltpu.VMEM((2, COMB_CH, D), jnp.float32),
            pltpu.SemaphoreType.DMA,
            pltpu.SemaphoreType.DMA,
            pltpu.SemaphoreType.DMA,
            pltpu.SemaphoreType.DMA,
        ],
    )
    def sc_combine(yg_hbm, z_hbm, s0_hbm, s1_hbm, out_hbm,
                   i0_v, i1_v, a_v, b_v, z_v, ls0, ls1, ws0, ws1):
        wid = lax.axis_index("s") * SC_NC + lax.axis_index("c")
        base0 = wid * COMB_PER_W
        pltpu.sync_copy(s0_hbm.at[pl.ds(base0, COMB_PER_W)], i0_v)
        pltpu.sync_copy(s1_hbm.at[pl.ds(base0, COMB_PER_W)], i1_v)
        lsem = (ls0, ls1)
        wsem = (ws0, ws1)

        def start_loads(c):
            buf = c % 2
            sl = pl.ds(c * COMB_CH, COMB_CH)
            return (
                pltpu.async_copy(yg_hbm.at[i0_v.at[sl]], a_v.at[buf], lsem[buf]),
                pltpu.async_copy(yg_hbm.at[i1_v.at[sl]], b_v.at[buf], lsem[buf]),
                pltpu.async_copy(z_hbm.at[pl.ds(base0 + c * COMB_CH, COMB_CH)],
                                 z_v.at[buf], lsem[buf]),
            )

        lh = {0: start_loads(0), 1: start_loads(1)}
        wh = {}
        for c in range(C_CHUNKS):
            buf = c % 2
            for h in lh[c]:
                h.wait()

            def row(i, _):
                def col(j, _):
                    for u in range(4):
                        sl = pl.ds(j * 64 + u * 16, 16)
                        z_v[buf, i, sl] = (z_v[buf, i, sl] + a_v[buf, i, sl]
                                           + b_v[buf, i, sl])
                    return 0

                lax.fori_loop(0, D // 64, col, 0)
                return 0

            lax.fori_loop(0, COMB_CH, row, 0)
            wh[c] = pltpu.async_copy(
                z_v.at[buf],
                out_hbm.at[pl.ds(base0 + c * COMB_CH, COMB_CH)], wsem[buf])
            if c + 2 < C_CHUNKS:
                wh[c].wait()
                lh[c + 2] = start_loads(c + 2)
        for c in range(max(0, C_CHUNKS - 2), C_CHUNKS):
            wh[c].wait()

    return sc_combine


def _sc_combine(yg, z, s0, s1):
    return _make_sc_combine()(yg, z, s0, s1)


def kernel(x, Wg, W1, W2, Wp, Ws1, Ws2, Wsp):
    Bq, Tq, C = x.shape
    xf = x.reshape(-1, C)
    wg_pad = jnp.pad(Wg, ((0, 0), (0, 128 - E)))
    scores_pad, idx_pad, w_pad = _gate(xf, wg_pad)
    scores = scores_pad[:, :E]
    e1 = idx_pad[:, 0]
    e2 = idx_pad[:, 1]
    w1 = w_pad[:, 0]
    w2 = w_pad[:, 1]

    pos, wf, be = _route(e1, e2, w1, w2)
    s0 = pos[0::2]
    s1 = pos[1::2]

    xbf = xf.astype(jnp.bfloat16)
    xpack = jax.lax.bitcast_convert_type(xbf.reshape(N, D // 2, 2), jnp.int32)
    xg_p, sw = _sc_gather(xpack, pos, wf)
    xg = jax.lax.bitcast_convert_type(xg_p, jnp.bfloat16).reshape(P_R, D)
    sw_full = jnp.broadcast_to(sw[:, None], (P_R, 128))
    yg = _ffn(be, xg, W1, W2, Wp, sw_full)
    z = _shared(xf, Ws1, Ws2, Wsp)
    out = _sc_combine(yg, z, s0, s1)
    return (out.reshape(Bq, Tq, C), scores)


# speedup vs baseline: 1.8335x; 1.8335x over previous
import functools

import jax
import jax.numpy as jnp
from jax import lax
from jax.experimental import pallas as pl
from jax.experimental.pallas import tpu as pltpu
from jax.experimental.pallas import tpu_sc as plsc

D = 1024
E = 8
K = 2
F_E = 1024
F_S = 2048
N = 2048

BLK = 128
P_R = N * K + E * BLK
NB = P_R // BLK

NEG = -1e30

SC_NC = 2
SC_NS = 16
SC_NW = SC_NC * SC_NS


def _gate_body(x_ref, wg_ref, s_ref, idx_ref, w_ref, xbf_ref):
    x = x_ref[...]
    xbf_ref[...] = x.astype(jnp.bfloat16)
    wg = wg_ref[...]
    s = jnp.dot(x, wg, preferred_element_type=jnp.float32)
    s_ref[...] = s
    lane = lax.broadcasted_iota(jnp.int32, s.shape, 1)
    sm = jnp.where(lane < E, s, NEG)
    m1 = jnp.max(sm, axis=1, keepdims=True)
    i1 = jnp.min(jnp.where(sm == m1, lane, 127), axis=1, keepdims=True)
    sm2 = jnp.where(lane == i1, NEG, sm)
    m2 = jnp.max(sm2, axis=1, keepdims=True)
    i2 = jnp.min(jnp.where(sm2 == m2, lane, 127), axis=1, keepdims=True)
    z = jnp.sum(jnp.exp(sm - m1), axis=1, keepdims=True)
    w1 = 1.0 / z
    w2 = jnp.exp(m2 - m1) / z
    idx_ref[...] = jnp.where(lane == 0, i1, jnp.where(lane == 1, i2, 0))
    w_ref[...] = jnp.where(lane == 0, w1, jnp.where(lane == 1, w2, 0.0))


def _gate(xf, wg_pad):
    tb = 256
    return pl.pallas_call(
        _gate_body,
        grid=(N // tb,),
        in_specs=[
            pl.BlockSpec((tb, D), lambda t: (t, 0)),
            pl.BlockSpec((D, 128), lambda t: (0, 0)),
        ],
        out_specs=[
            pl.BlockSpec((tb, 128), lambda t: (t, 0)),
            pl.BlockSpec((tb, 128), lambda t: (t, 0)),
            pl.BlockSpec((tb, 128), lambda t: (t, 0)),
            pl.BlockSpec((tb, D), lambda t: (t, 0)),
        ],
        out_shape=[
            jax.ShapeDtypeStruct((N, 128), jnp.float32),
            jax.ShapeDtypeStruct((N, 128), jnp.int32),
            jax.ShapeDtypeStruct((N, 128), jnp.float32),
            jax.ShapeDtypeStruct((N, D), jnp.bfloat16),
        ],
    )(xf, wg_pad)


def _route(e1, e2, w1, w2):
    ef = jnp.stack([e1, e2], axis=1).reshape(-1)
    wf = jnp.stack([w1, w2], axis=1).reshape(-1)
    onehot = (ef[:, None] == jnp.arange(E, dtype=jnp.int32)[None, :])
    counts = jnp.sum(onehot, axis=0, dtype=jnp.int32)
    pc = ((counts + BLK - 1) // BLK) * BLK
    ends = jnp.cumsum(pc)
    gs = ends - pc
    rank = jnp.cumsum(onehot.astype(jnp.int32), axis=0) - 1
    r = jnp.sum(rank * onehot, axis=1)
    pos = gs[ef] + r
    b_start = jnp.arange(NB, dtype=jnp.int32) * BLK
    be = jnp.sum(b_start[:, None] >= ends[None, :], axis=1).astype(jnp.int32)
    be = jnp.minimum(be, E - 1)
    return pos, wf, be


GATHER_PER_W = P_R // SC_NW
GATHER_CH = 16
G_CHUNKS = GATHER_PER_W // GATHER_CH
G_NBUF = 4


@functools.lru_cache(maxsize=None)
def _make_sc_gather():
    mesh = plsc.VectorSubcoreMesh(core_axis_name="c", subcore_axis_name="s")

    @functools.partial(
        pl.kernel,
        mesh=mesh,
        out_type=(
            jax.ShapeDtypeStruct((P_R, D), jnp.float32),
            jax.ShapeDtypeStruct((P_R,), jnp.float32),
        ),
        scratch_types=[
            pltpu.VMEM((N * K,), jnp.int32),
            pltpu.VMEM((N * K,), jnp.float32),
            pltpu.VMEM((GATHER_PER_W,), jnp.int32),
            pltpu.VMEM((GATHER_PER_W,), jnp.float32),
            pltpu.VMEM((G_NBUF, GATHER_CH, D), jnp.float32),
        ] + [pltpu.SemaphoreType.DMA] * (2 * G_NBUF),
        compiler_params=pltpu.CompilerParams(needs_layout_passes=False),
    )
    def sc_gather(x_hbm, pos_hbm, wf_hbm, out_hbm, sw_hbm,
                  pos_v, wf_v, gidx_v, sw_v, rows_v, *sems):
        gsem = sems[:G_NBUF]
        wsem = sems[G_NBUF:]
        wid = lax.axis_index("s") * SC_NC + lax.axis_index("c")
        base0 = wid * GATHER_PER_W
        pltpu.sync_copy(pos_hbm, pos_v)
        pltpu.sync_copy(wf_hbm, wf_v)
        lane = lax.iota(jnp.int32, 16)

        def init_body(i, _):
            sl = pl.ds(i * 16, 16)
            gidx_v[sl] = (base0 + i * 16 + lane) & (N - 1)
            sw_v[sl] = jnp.zeros((16,), jnp.float32)
            return 0

        lax.fori_loop(0, GATHER_PER_W // 16, init_body, 0)

        def scan_body(i, _):
            s = i * 16
            p = pos_v[pl.ds(s, 16)]
            w = wf_v[pl.ds(s, 16)]
            tok = (s + lane) >> 1
            li = p - base0
            m = (li >= 0) & (li < GATHER_PER_W)
            li = jnp.clip(li, 0, GATHER_PER_W - 1)
            plsc.store_scatter(gidx_v, [li], tok, mask=m)
            plsc.store_scatter(sw_v, [li], w, mask=m)
            return 0

        lax.fori_loop(0, (N * K) // 16, scan_body, 0)
        pltpu.sync_copy(sw_v, sw_hbm.at[pl.ds(base0, GATHER_PER_W)])

        def start_gather(c):
            buf = c % G_NBUF
            return pltpu.async_copy(
                x_hbm.at[gidx_v.at[pl.ds(c * GATHER_CH, GATHER_CH)]],
                rows_v.at[buf], gsem[buf])

        gh = {c: start_gather(c) for c in range(G_NBUF)}
        wh = {}
        for c in range(G_CHUNKS):
            buf = c % G_NBUF
            gh[c].wait()
            wh[c] = pltpu.async_copy(
                rows_v.at[buf],
                out_hbm.at[pl.ds(base0 + c * GATHER_CH, GATHER_CH)],
                wsem[buf])
            if c + G_NBUF < G_CHUNKS:
                wh[c].wait()
                gh[c + G_NBUF] = start_gather(c + G_NBUF)
        for c in range(max(0, G_CHUNKS - G_NBUF), G_CHUNKS):
            wh[c].wait()

    return sc_gather


def _sc_gather(xbf, pos, wf):
    return _make_sc_gather()(xbf, pos, wf)


def _ffn_body(be_ref, xg_ref, w1_ref, w2_ref, wp_ref, sw_ref, out_ref):
    xb = xg_ref[...].astype(jnp.bfloat16)
    w1 = w1_ref[0].astype(jnp.bfloat16)
    w2 = w2_ref[0].astype(jnp.bfloat16)
    h1 = jnp.dot(xb, w1, preferred_element_type=jnp.float32)
    h2 = jnp.dot(xb, w2, preferred_element_type=jnp.float32)
    h = ((h1 * jax.nn.sigmoid(h1)) * h2).astype(jnp.bfloat16)
    wp = wp_ref[0].astype(jnp.bfloat16)
    y = jnp.dot(h, wp, preferred_element_type=jnp.float32)
    sw = sw_ref[...][:, 0:1]
    out_ref[...] = y * sw


def _ffn(be, xg, w1, w2, wp, sw_full):
    spec = pltpu.PrefetchScalarGridSpec(
        num_scalar_prefetch=1,
        grid=(NB,),
        in_specs=[
            pl.BlockSpec((BLK, D), lambda b, be: (b, 0)),
            pl.BlockSpec((1, D, F_E), lambda b, be: (be[b], 0, 0)),
            pl.BlockSpec((1, D, F_E), lambda b, be: (be[b], 0, 0)),
            pl.BlockSpec((1, F_E, D), lambda b, be: (be[b], 0, 0)),
            pl.BlockSpec((BLK, 128), lambda b, be: (b, 0)),
        ],
        out_specs=pl.BlockSpec((BLK, D), lambda b, be: (b, 0)),
    )
    return pl.pallas_call(
        _ffn_body,
        grid_spec=spec,
        out_shape=jax.ShapeDtypeStruct((P_R, D), jnp.float32),
    )(be, xg, w1, w2, wp, sw_full)


F_T = 512


def _shared_body(x_ref, ws1_ref, ws2_ref, wsp_ref, z_ref):
    f = pl.program_id(0)
    x = x_ref[...]
    ws1 = ws1_ref[...].astype(jnp.bfloat16)
    ws2 = ws2_ref[...].astype(jnp.bfloat16)
    h1 = jnp.dot(x, ws1, preferred_element_type=jnp.float32)
    h2 = jnp.dot(x, ws2, preferred_element_type=jnp.float32)
    h = ((h1 * jax.nn.sigmoid(h1)) * h2).astype(jnp.bfloat16)
    wsp = wsp_ref[...].astype(jnp.bfloat16)
    part = jnp.dot(h, wsp, preferred_element_type=jnp.float32)

    @pl.when(f == 0)
    def _():
        z_ref[...] = part

    @pl.when(f != 0)
    def _():
        z_ref[...] = z_ref[...] + part


def _shared(xf, ws1, ws2, wsp):
    return pl.pallas_call(
        _shared_body,
        grid=(F_S // F_T,),
        in_specs=[
            pl.BlockSpec((N, D), lambda f: (0, 0)),
            pl.BlockSpec((D, F_T), lambda f: (0, f)),
            pl.BlockSpec((D, F_T), lambda f: (0, f)),
            pl.BlockSpec((F_T, D), lambda f: (f, 0)),
        ],
        out_specs=pl.BlockSpec((N, D), lambda f: (0, 0)),
        out_shape=jax.ShapeDtypeStruct((N, D), jnp.float32),
    )(xf, ws1, ws2, wsp)


COMB_PER_W = N // SC_NW
COMB_CH = 16
C_CHUNKS = COMB_PER_W // COMB_CH


@functools.lru_cache(maxsize=None)
def _make_sc_combine():
    mesh = plsc.VectorSubcoreMesh(core_axis_name="c", subcore_axis_name="s")

    @functools.partial(
        pl.kernel,
        mesh=mesh,
        out_type=jax.ShapeDtypeStruct((N, D), jnp.float32),
        scratch_types=[
            pltpu.VMEM((COMB_PER_W,), jnp.int32),
            pltpu.VMEM((COMB_PER_W,), jnp.int32),
            pltpu.VMEM((2, COMB_CH, D), jnp.float32),
            pltpu.VMEM((2, COMB_CH, D), jnp.float32),
            pltpu.VMEM((2, COMB_CH, D), jnp.float32),
            pltpu.SemaphoreType.DMA,
            pltpu.SemaphoreType.DMA,
            pltpu.SemaphoreType.DMA,
            pltpu.SemaphoreType.DMA,
        ],
    )
    def sc_combine(yg_hbm, z_hbm, s0_hbm, s1_hbm, out_hbm,
                   i0_v, i1_v, a_v, b_v, z_v, ls0, ls1, ws0, ws1):
        wid = lax.axis_index("s") * SC_NC + lax.axis_index("c")
        base0 = wid * COMB_PER_W
        pltpu.sync_copy(s0_hbm.at[pl.ds(base0, COMB_PER_W)], i0_v)
        pltpu.sync_copy(s1_hbm.at[pl.ds(base0, COMB_PER_W)], i1_v)
        lsem = (ls0, ls1)
        wsem = (ws0, ws1)

        def start_loads(c):
            buf = c % 2
            sl = pl.ds(c * COMB_CH, COMB_CH)
            return (
                pltpu.async_copy(yg_hbm.at[i0_v.at[sl]], a_v.at[buf], lsem[buf]),
                pltpu.async_copy(yg_hbm.at[i1_v.at[sl]], b_v.at[buf], lsem[buf]),
                pltpu.async_copy(z_hbm.at[pl.ds(base0 + c * COMB_CH, COMB_CH)],
                                 z_v.at[buf], lsem[buf]),
            )

        lh = {0: start_loads(0), 1: start_loads(1)}
        wh = {}
        for c in range(C_CHUNKS):
            buf = c % 2
            for h in lh[c]:
                h.wait()

            def row(i, _):
                def col(j, _):
                    for u in range(4):
                        sl = pl.ds(j * 64 + u * 16, 16)
                        z_v[buf, i, sl] = (z_v[buf, i, sl] + a_v[buf, i, sl]
                                           + b_v[buf, i, sl])
                    return 0

                lax.fori_loop(0, D // 64, col, 0)
                return 0

            lax.fori_loop(0, COMB_CH, row, 0)
            wh[c] = pltpu.async_copy(
                z_v.at[buf],
                out_hbm.at[pl.ds(base0 + c * COMB_CH, COMB_CH)], wsem[buf])
            if c + 2 < C_CHUNKS:
                wh[c].wait()
                lh[c + 2] = start_loads(c + 2)
        for c in range(max(0, C_CHUNKS - 2), C_CHUNKS):
            wh[c].wait()

    return sc_combine


def _sc_combine(yg, z, s0, s1):
    return _make_sc_combine()(yg, z, s0, s1)


def kernel(x, Wg, W1, W2, Wp, Ws1, Ws2, Wsp):
    Bq, Tq, C = x.shape
    xf = x.reshape(-1, C)
    wg_pad = jnp.pad(Wg, ((0, 0), (0, 128 - E)))
    scores_pad, idx_pad, w_pad, xbf = _gate(xf, wg_pad)
    scores = scores_pad[:, :E]
    e1 = idx_pad[:, 0]
    e2 = idx_pad[:, 1]
    w1 = w_pad[:, 0]
    w2 = w_pad[:, 1]

    pos, wf, be = _route(e1, e2, w1, w2)
    s0 = pos[0::2]
    s1 = pos[1::2]

    xg, sw = _sc_gather(xf, pos, wf)
    sw_full = jnp.broadcast_to(sw[:, None], (P_R, 128))
    yg = _ffn(be, xg, W1, W2, Wp, sw_full)
    z = _shared(xbf, Ws1, Ws2, Wsp)
    out = _sc_combine(yg, z, s0, s1)
    return (out.reshape(Bq, Tq, C), scores)


# speedup vs baseline: 1.9278x; 1.0515x over previous
import functools

import jax
import jax.numpy as jnp
from jax import lax
from jax.experimental import pallas as pl
from jax.experimental.pallas import tpu as pltpu
from jax.experimental.pallas import tpu_sc as plsc

D = 1024
E = 8
K = 2
F_E = 1024
F_S = 2048
N = 2048

BLK = 128
P_R = N * K + E * BLK
NB = P_R // BLK

NEG = -1e30

SC_NC = 2
SC_NS = 16
SC_NW = SC_NC * SC_NS


def _gate_body(x_ref, wg_ref, s_ref, idx_ref, w_ref, xbf_ref):
    x = x_ref[...]
    xbf_ref[...] = x.astype(jnp.bfloat16)
    wg = wg_ref[...]
    s = jnp.dot(x, wg, preferred_element_type=jnp.float32)
    s_ref[...] = s
    lane = lax.broadcasted_iota(jnp.int32, s.shape, 1)
    sm = jnp.where(lane < E, s, NEG)
    m1 = jnp.max(sm, axis=1, keepdims=True)
    i1 = jnp.min(jnp.where(sm == m1, lane, 127), axis=1, keepdims=True)
    sm2 = jnp.where(lane == i1, NEG, sm)
    m2 = jnp.max(sm2, axis=1, keepdims=True)
    i2 = jnp.min(jnp.where(sm2 == m2, lane, 127), axis=1, keepdims=True)
    z = jnp.sum(jnp.exp(sm - m1), axis=1, keepdims=True)
    w1 = 1.0 / z
    w2 = jnp.exp(m2 - m1) / z
    idx_ref[...] = jnp.where(lane == 0, i1, jnp.where(lane == 1, i2, 0))
    w_ref[...] = jnp.where(lane == 0, w1, jnp.where(lane == 1, w2, 0.0))


def _gate(xf, wg_pad):
    tb = 256
    return pl.pallas_call(
        _gate_body,
        grid=(N // tb,),
        in_specs=[
            pl.BlockSpec((tb, D), lambda t: (t, 0)),
            pl.BlockSpec((D, 128), lambda t: (0, 0)),
        ],
        out_specs=[
            pl.BlockSpec((tb, 128), lambda t: (t, 0)),
            pl.BlockSpec((tb, 128), lambda t: (t, 0)),
            pl.BlockSpec((tb, 128), lambda t: (t, 0)),
            pl.BlockSpec((tb, D), lambda t: (t, 0)),
        ],
        out_shape=[
            jax.ShapeDtypeStruct((N, 128), jnp.float32),
            jax.ShapeDtypeStruct((N, 128), jnp.int32),
            jax.ShapeDtypeStruct((N, 128), jnp.float32),
            jax.ShapeDtypeStruct((N, D), jnp.bfloat16),
        ],
    )(xf, wg_pad)


def _route(e1, e2):
    er = jnp.arange(E, dtype=jnp.int32)[None, :]
    oh1 = e1[:, None] == er
    oh2 = e2[:, None] == er
    oh = oh1.astype(jnp.int32) + oh2.astype(jnp.int32)
    cum = jnp.cumsum(oh, axis=0)
    counts = cum[-1]
    cumex = cum - oh
    pc = ((counts + BLK - 1) // BLK) * BLK
    ends = jnp.cumsum(pc)
    gs = ends - pc
    base = cumex + gs[None, :]
    pos0 = jnp.sum(jnp.where(oh1, base, 0), axis=1)
    pos1 = jnp.sum(jnp.where(oh2, base + oh1.astype(jnp.int32), 0), axis=1)
    b_start = jnp.arange(NB, dtype=jnp.int32) * BLK
    be = jnp.sum(b_start[:, None] >= ends[None, :], axis=1).astype(jnp.int32)
    be = jnp.minimum(be, E - 1)
    return pos0, pos1, be


GATHER_PER_W = P_R // SC_NW
GATHER_CH = 16
G_CHUNKS = GATHER_PER_W // GATHER_CH
G_NBUF = 4


@functools.lru_cache(maxsize=None)
def _make_sc_gather():
    mesh = plsc.VectorSubcoreMesh(core_axis_name="c", subcore_axis_name="s")

    @functools.partial(
        pl.kernel,
        mesh=mesh,
        out_type=(
            jax.ShapeDtypeStruct((P_R, D), jnp.float32),
            jax.ShapeDtypeStruct((P_R,), jnp.float32),
        ),
        scratch_types=[
            pltpu.VMEM((N,), jnp.int32),
            pltpu.VMEM((N,), jnp.int32),
            pltpu.VMEM((N,), jnp.float32),
            pltpu.VMEM((N,), jnp.float32),
            pltpu.VMEM((GATHER_PER_W,), jnp.int32),
            pltpu.VMEM((GATHER_PER_W,), jnp.float32),
            pltpu.VMEM((G_NBUF, GATHER_CH, D), jnp.float32),
        ] + [pltpu.SemaphoreType.DMA] * (2 * G_NBUF),
        compiler_params=pltpu.CompilerParams(needs_layout_passes=False),
    )
    def sc_gather(x_hbm, p0_hbm, p1_hbm, w0_hbm, w1_hbm, out_hbm, sw_hbm,
                  p0_v, p1_v, w0_v, w1_v, gidx_v, sw_v, rows_v, *sems):
        gsem = sems[:G_NBUF]
        wsem = sems[G_NBUF:]
        wid = lax.axis_index("s") * SC_NC + lax.axis_index("c")
        base0 = wid * GATHER_PER_W
        pltpu.sync_copy(p0_hbm, p0_v)
        pltpu.sync_copy(p1_hbm, p1_v)
        pltpu.sync_copy(w0_hbm, w0_v)
        pltpu.sync_copy(w1_hbm, w1_v)
        lane = lax.iota(jnp.int32, 16)

        def init_body(i, _):
            sl = pl.ds(i * 16, 16)
            gidx_v[sl] = (base0 + i * 16 + lane) & (N - 1)
            sw_v[sl] = jnp.zeros((16,), jnp.float32)
            return 0

        lax.fori_loop(0, GATHER_PER_W // 16, init_body, 0)

        def scan_body(i, _):
            s = i * 16
            tok = s + lane
            for pv, wv in ((p0_v, w0_v), (p1_v, w1_v)):
                p = pv[pl.ds(s, 16)]
                w = wv[pl.ds(s, 16)]
                li = p - base0
                m = (li >= 0) & (li < GATHER_PER_W)
                li = jnp.clip(li, 0, GATHER_PER_W - 1)
                plsc.store_scatter(gidx_v, [li], tok, mask=m)
                plsc.store_scatter(sw_v, [li], w, mask=m)
            return 0

        lax.fori_loop(0, N // 16, scan_body, 0)
        pltpu.sync_copy(sw_v, sw_hbm.at[pl.ds(base0, GATHER_PER_W)])

        def start_gather(c):
            buf = c % G_NBUF
            return pltpu.async_copy(
                x_hbm.at[gidx_v.at[pl.ds(c * GATHER_CH, GATHER_CH)]],
                rows_v.at[buf], gsem[buf])

        gh = {c: start_gather(c) for c in range(G_NBUF)}
        wh = {}
        for c in range(G_CHUNKS):
            buf = c % G_NBUF
            gh[c].wait()
            wh[c] = pltpu.async_copy(
                rows_v.at[buf],
                out_hbm.at[pl.ds(base0 + c * GATHER_CH, GATHER_CH)],
                wsem[buf])
            if c + G_NBUF < G_CHUNKS:
                wh[c].wait()
                gh[c + G_NBUF] = start_gather(c + G_NBUF)
        for c in range(max(0, G_CHUNKS - G_NBUF), G_CHUNKS):
            wh[c].wait()

    return sc_gather


def _sc_gather(xf, pos0, pos1, w0, w1):
    return _make_sc_gather()(xf, pos0, pos1, w0, w1)


def _ffn_body(be_ref, xg_ref, w1_ref, w2_ref, wp_ref, sw_ref, out_ref):
    xb = xg_ref[...].astype(jnp.bfloat16)
    w1 = w1_ref[0].astype(jnp.bfloat16)
    w2 = w2_ref[0].astype(jnp.bfloat16)
    h1 = jnp.dot(xb, w1, preferred_element_type=jnp.float32)
    h2 = jnp.dot(xb, w2, preferred_element_type=jnp.float32)
    h = ((h1 * jax.nn.sigmoid(h1)) * h2).astype(jnp.bfloat16)
    wp = wp_ref[0].astype(jnp.bfloat16)
    y = jnp.dot(h, wp, preferred_element_type=jnp.float32)
    sw = sw_ref[...][:, 0:1]
    out_ref[...] = y * sw


def _ffn(be, xg, w1, w2, wp, sw_full):
    spec = pltpu.PrefetchScalarGridSpec(
        num_scalar_prefetch=1,
        grid=(NB,),
        in_specs=[
            pl.BlockSpec((BLK, D), lambda b, be: (b, 0)),
            pl.BlockSpec((1, D, F_E), lambda b, be: (be[b], 0, 0)),
            pl.BlockSpec((1, D, F_E), lambda b, be: (be[b], 0, 0)),
            pl.BlockSpec((1, F_E, D), lambda b, be: (be[b], 0, 0)),
            pl.BlockSpec((BLK, 128), lambda b, be: (b, 0)),
        ],
        out_specs=pl.BlockSpec((BLK, D), lambda b, be: (b, 0)),
    )
    return pl.pallas_call(
        _ffn_body,
        grid_spec=spec,
        out_shape=jax.ShapeDtypeStruct((P_R, D), jnp.float32),
    )(be, xg, w1, w2, wp, sw_full)


F_T = 512


def _shared_body(x_ref, ws1_ref, ws2_ref, wsp_ref, z_ref):
    f = pl.program_id(0)
    x = x_ref[...]
    ws1 = ws1_ref[...].astype(jnp.bfloat16)
    ws2 = ws2_ref[...].astype(jnp.bfloat16)
    h1 = jnp.dot(x, ws1, preferred_element_type=jnp.float32)
    h2 = jnp.dot(x, ws2, preferred_element_type=jnp.float32)
    h = ((h1 * jax.nn.sigmoid(h1)) * h2).astype(jnp.bfloat16)
    wsp = wsp_ref[...].astype(jnp.bfloat16)
    part = jnp.dot(h, wsp, preferred_element_type=jnp.float32)

    @pl.when(f == 0)
    def _():
        z_ref[...] = part

    @pl.when(f != 0)
    def _():
        z_ref[...] = z_ref[...] + part


def _shared(xf, ws1, ws2, wsp):
    return pl.pallas_call(
        _shared_body,
        grid=(F_S // F_T,),
        in_specs=[
            pl.BlockSpec((N, D), lambda f: (0, 0)),
            pl.BlockSpec((D, F_T), lambda f: (0, f)),
            pl.BlockSpec((D, F_T), lambda f: (0, f)),
            pl.BlockSpec((F_T, D), lambda f: (f, 0)),
        ],
        out_specs=pl.BlockSpec((N, D), lambda f: (0, 0)),
        out_shape=jax.ShapeDtypeStruct((N, D), jnp.float32),
    )(xf, ws1, ws2, wsp)


COMB_PER_W = N // SC_NW
COMB_CH = 16
C_CHUNKS = COMB_PER_W // COMB_CH


@functools.lru_cache(maxsize=None)
def _make_sc_combine():
    mesh = plsc.VectorSubcoreMesh(core_axis_name="c", subcore_axis_name="s")

    @functools.partial(
        pl.kernel,
        mesh=mesh,
        out_type=jax.ShapeDtypeStruct((N, D), jnp.float32),
        scratch_types=[
            pltpu.VMEM((COMB_PER_W,), jnp.int32),
            pltpu.VMEM((COMB_PER_W,), jnp.int32),
            pltpu.VMEM((2, COMB_CH, D), jnp.float32),
            pltpu.VMEM((2, COMB_CH, D), jnp.float32),
            pltpu.VMEM((2, COMB_CH, D), jnp.float32),
            pltpu.SemaphoreType.DMA,
            pltpu.SemaphoreType.DMA,
            pltpu.SemaphoreType.DMA,
            pltpu.SemaphoreType.DMA,
        ],
    )
    def sc_combine(yg_hbm, z_hbm, s0_hbm, s1_hbm, out_hbm,
                   i0_v, i1_v, a_v, b_v, z_v, ls0, ls1, ws0, ws1):
        wid = lax.axis_index("s") * SC_NC + lax.axis_index("c")
        base0 = wid * COMB_PER_W
        pltpu.sync_copy(s0_hbm.at[pl.ds(base0, COMB_PER_W)], i0_v)
        pltpu.sync_copy(s1_hbm.at[pl.ds(base0, COMB_PER_W)], i1_v)
        lsem = (ls0, ls1)
        wsem = (ws0, ws1)

        def start_loads(c):
            buf = c % 2
            sl = pl.ds(c * COMB_CH, COMB_CH)
            return (
                pltpu.async_copy(yg_hbm.at[i0_v.at[sl]], a_v.at[buf], lsem[buf]),
                pltpu.async_copy(yg_hbm.at[i1_v.at[sl]], b_v.at[buf], lsem[buf]),
                pltpu.async_copy(z_hbm.at[pl.ds(base0 + c * COMB_CH, COMB_CH)],
                                 z_v.at[buf], lsem[buf]),
            )

        lh = {0: start_loads(0), 1: start_loads(1)}
        wh = {}
        for c in range(C_CHUNKS):
            buf = c % 2
            for h in lh[c]:
                h.wait()

            def row(i, _):
                def col(j, _):
                    for u in range(4):
                        sl = pl.ds(j * 64 + u * 16, 16)
                        z_v[buf, i, sl] = (z_v[buf, i, sl] + a_v[buf, i, sl]
                                           + b_v[buf, i, sl])
                    return 0

                lax.fori_loop(0, D // 64, col, 0)
                return 0

            lax.fori_loop(0, COMB_CH, row, 0)
            wh[c] = pltpu.async_copy(
                z_v.at[buf],
                out_hbm.at[pl.ds(base0 + c * COMB_CH, COMB_CH)], wsem[buf])
            if c + 2 < C_CHUNKS:
                wh[c].wait()
                lh[c + 2] = start_loads(c + 2)
        for c in range(max(0, C_CHUNKS - 2), C_CHUNKS):
            wh[c].wait()

    return sc_combine


def _sc_combine(yg, z, s0, s1):
    return _make_sc_combine()(yg, z, s0, s1)


def kernel(x, Wg, W1, W2, Wp, Ws1, Ws2, Wsp):
    Bq, Tq, C = x.shape
    xf = x.reshape(-1, C)
    wg_pad = jnp.pad(Wg, ((0, 0), (0, 128 - E)))
    scores_pad, idx_pad, w_pad, xbf = _gate(xf, wg_pad)
    scores = scores_pad[:, :E]
    e1 = idx_pad[:, 0]
    e2 = idx_pad[:, 1]
    w1 = w_pad[:, 0]
    w2 = w_pad[:, 1]

    pos0, pos1, be = _route(e1, e2)

    xg, sw = _sc_gather(xf, pos0, pos1, w1, w2)
    sw_full = jnp.broadcast_to(sw[:, None], (P_R, 128))
    yg = _ffn(be, xg, W1, W2, Wp, sw_full)
    z = _shared(xbf, Ws1, Ws2, Wsp)
    out = _sc_combine(yg, z, pos0, pos1)
    return (out.reshape(Bq, Tq, C), scores)
